# Initial kernel scaffold; baseline (speedup 1.0000x reference)
import jax
import jax.numpy as jnp
from jax.experimental import pallas as pl


def kernel(x, target_id, embed_w, embed_out_w, fc1_w, fc1_b):
    raise NotImplementedError("write your pallas kernel here")


# speedup vs baseline: 2.1220x; 2.1220x over previous
import functools

import jax
import jax.numpy as jnp
from jax import lax
from jax.experimental import pallas as pl
from jax.experimental.pallas import tpu as pltpu
from jax.experimental.pallas import tpu_sc as plsc

HORSE_COUNT = 1000000
NDIM = 16
BATCH = 16384
CTX = 6

NC = 2
NS = 16
NW = NC * NS

CHUNK = 128
G_TOTAL = BATCH * CTX
G_CHUNKS = G_TOTAL // (NW * CHUNK)
T_CHUNKS = BATCH // (NW * CHUNK)

def _sc_gather_body(x_ref, tid_ref, emb_ref, embo_ref, g_ref, t_ref,
                    idx_v, rows_v, tid_v, trow_v, sem_g, sem_t):
    c = lax.axis_index("c")
    s = lax.axis_index("s")
    w = s * NC + c
    pltpu.sync_copy(x_ref.at[pl.ds(w * G_CHUNKS, G_CHUNKS)], idx_v)
    pltpu.sync_copy(tid_ref.at[pl.ds(w * T_CHUNKS, T_CHUNKS)], tid_v)
    gcp = [pltpu.async_copy(emb_ref.at[idx_v.at[j]], rows_v.at[j], sem_g)
           for j in range(G_CHUNKS)]
    tcp = [pltpu.async_copy(embo_ref.at[tid_v.at[j]], trow_v.at[j], sem_t)
           for j in range(T_CHUNKS)]
    for cp in gcp:
        cp.wait()
    pltpu.sync_copy(rows_v, g_ref.at[pl.ds(w * G_CHUNKS, G_CHUNKS)])
    for cp in tcp:
        cp.wait()
    pltpu.sync_copy(trow_v, t_ref.at[pl.ds(w * T_CHUNKS, T_CHUNKS)])


@functools.cache
def _sc_gather():
    mesh = plsc.VectorSubcoreMesh(
        core_axis_name="c", subcore_axis_name="s", num_cores=NC, num_subcores=NS
    )
    return pl.kernel(
        _sc_gather_body,
        out_type=(
            jax.ShapeDtypeStruct((G_TOTAL // CHUNK, CHUNK, NDIM), jnp.float32),
            jax.ShapeDtypeStruct((BATCH // CHUNK, CHUNK, NDIM), jnp.float32),
        ),
        mesh=mesh,
        scratch_types=(
            pltpu.VMEM((G_CHUNKS, CHUNK), jnp.int32),
            pltpu.VMEM((G_CHUNKS, CHUNK, NDIM), jnp.float32),
            pltpu.VMEM((T_CHUNKS, CHUNK), jnp.int32),
            pltpu.VMEM((T_CHUNKS, CHUNK, NDIM), jnp.float32),
            pltpu.SemaphoreType.DMA,
            pltpu.SemaphoreType.DMA,
        ),
        compiler_params=pltpu.CompilerParams(use_tc_tiling_on_sc=False),
    )


def _tc_dense(g_ref, t_ref, w_ref, b_ref, o_ref):
    g = g_ref[...]
    w = w_ref[...]
    acc = lax.dot_general(g, w, (((1,), (1,)), ((), ())),
                          preferred_element_type=jnp.float32)
    acc = acc + b_ref[...]
    o = jnp.maximum(acc, 0.0)
    a = jnp.sum(o * t_ref[...], axis=1)
    o_ref[...] = 1.0 / (1.0 + jnp.exp(-a))


_tc_call = pl.pallas_call(
    _tc_dense,
    out_shape=jax.ShapeDtypeStruct((BATCH,), jnp.float32),
)


def kernel(x, target_id, embed_w, embed_out_w, fc1_w, fc1_b):
    x2d = x.reshape(G_TOTAL // CHUNK, CHUNK)
    tid2d = target_id.reshape(BATCH // CHUNK, CHUNK)
    g3, t3 = _sc_gather()(x2d, tid2d, embed_w, embed_out_w)
    g = g3.reshape(BATCH, CTX * NDIM)
    t = t3.reshape(BATCH, NDIM)
    return _tc_call(g, t, fc1_w, fc1_b.reshape(1, NDIM))
